# baseline (device time: 134466 ns/iter reference)
import jax
import jax.numpy as jnp
from jax import lax
from jax.experimental import pallas as pl
from jax.experimental.pallas import tpu as pltpu

N_DEV = 4
B, Sq, Skv, Dh = 2, 512, 512, 64
H_LOC = 8
D_LOC = H_LOC * Dh
D_MODEL = 768
WINDOW = 128


def kernel(x, Wq, K_ext, V_ext, Wo):
    def body(x_ref, wq_ref, k_ref, v_ref, wo_ref, out_ref,
             comm_ref, send_sems, recv_sems):
        my_pos = lax.axis_index("i")
        left = (my_pos - 1) % N_DEV
        right = (my_pos + 1) % N_DEV

        barrier_sem = pltpu.get_barrier_semaphore()
        for nbr in [left, right]:
            pl.semaphore_signal(
                barrier_sem, inc=1,
                device_id=(nbr,), device_id_type=pl.DeviceIdType.MESH,
            )
        pl.semaphore_wait(barrier_sem, 2)

        col0 = my_pos * D_LOC

        qi = lax.broadcasted_iota(jnp.int32, (Sq, Skv), 0)
        ki = lax.broadcasted_iota(jnp.int32, (Sq, Skv), 1)
        neg = jnp.where(jnp.abs(qi - ki) <= WINDOW,
                        jnp.float32(0.0), jnp.float32(-1e9))

        wq_loc = wq_ref[:, pl.ds(col0, D_LOC)]
        wo_loc = wo_ref[pl.ds(col0, D_LOC), :]

        for b in range(B):
            qb = jnp.dot(x_ref[b], wq_loc,
                         preferred_element_type=jnp.float32)
            ctx_cols = []
            for h in range(H_LOC):
                q = qb[:, h * Dh:(h + 1) * Dh]
                k = k_ref[b, :, h, :]
                v = v_ref[b, :, h, :]
                s = lax.dot_general(
                    q, k, (((1,), (1,)), ((), ())),
                    preferred_element_type=jnp.float32) * 0.125
                s = s + neg
                m = jnp.max(s, axis=1, keepdims=True)
                w = jnp.exp(s - m)
                w = w / jnp.sum(w, axis=1, keepdims=True)
                ctx_cols.append(jnp.dot(w, v,
                                        preferred_element_type=jnp.float32))
            ctx_b = jnp.concatenate(ctx_cols, axis=1)
            partial_b = jnp.dot(ctx_b, wo_loc,
                                preferred_element_type=jnp.float32)
            out_ref[b] = partial_b
            comm_ref[0, b] = partial_b

        for h in range(N_DEV - 1):
            rdma = pltpu.make_async_remote_copy(
                src_ref=comm_ref.at[h],
                dst_ref=comm_ref.at[h + 1],
                send_sem=send_sems.at[h],
                recv_sem=recv_sems.at[h],
                device_id=(right,),
                device_id_type=pl.DeviceIdType.MESH,
            )
            rdma.start()
            rdma.wait()
            out_ref[...] += comm_ref[h + 1]

    return pl.pallas_call(
        body,
        out_shape=jax.ShapeDtypeStruct((B, Sq, D_MODEL), jnp.float32),
        in_specs=[pl.BlockSpec(memory_space=pltpu.VMEM)] * 5,
        out_specs=pl.BlockSpec(memory_space=pltpu.VMEM),
        scratch_shapes=[
            pltpu.VMEM((N_DEV, B, Sq, D_MODEL), jnp.float32),
            pltpu.SemaphoreType.DMA((N_DEV - 1,)),
            pltpu.SemaphoreType.DMA((N_DEV - 1,)),
        ],
        compiler_params=pltpu.CompilerParams(collective_id=0),
    )(x, Wq, K_ext, V_ext, Wo)


# device time: 60324 ns/iter; 2.2291x vs baseline; 2.2291x over previous
import jax
import jax.numpy as jnp
from jax import lax
from jax.experimental import pallas as pl
from jax.experimental.pallas import tpu as pltpu

N_DEV = 4
B, Sq, Skv, Dh = 2, 512, 512, 64
H_LOC = 8
D_LOC = H_LOC * Dh
D_MODEL = 768
HALF = D_MODEL // 2
CH = Sq // N_DEV
WINDOW = 128
N_STEP = 2 * (N_DEV - 1)

COMM_DT = jnp.float32


def kernel(x, Wq, K_ext, V_ext, Wo):
    def body(x_ref, wq_ref, k_ref, v_ref, wo_ref, out_ref, part_ref,
             stage_r, recv_r, stage_l, recv_l,
             ssem_r, rsem_r, ssem_l, rsem_l):
        p = lax.axis_index("i")
        left = (p - 1) % N_DEV
        right = (p + 1) % N_DEV

        barrier_sem = pltpu.get_barrier_semaphore()
        for nbr in [left, right]:
            pl.semaphore_signal(
                barrier_sem, inc=1,
                device_id=(nbr,), device_id_type=pl.DeviceIdType.MESH,
            )
        pl.semaphore_wait(barrier_sem, 2)

        col0 = p * D_LOC
        wq_loc = wq_ref[:, pl.ds(col0, D_LOC)]
        wo_loc = wo_ref[pl.ds(col0, D_LOC), :]

        qi = lax.broadcasted_iota(jnp.int32, (Sq, Skv), 0)
        ki = lax.broadcasted_iota(jnp.int32, (Sq, Skv), 1)
        mask01 = jnp.where(jnp.abs(qi - ki) <= WINDOW,
                           jnp.float32(1.0), jnp.float32(0.0))

        x2 = x_ref[...].reshape(B * Sq, D_MODEL)
        q_all = jnp.dot(x2, wq_loc,
                        preferred_element_type=jnp.float32) * 0.125

        ctx_rows = []
        for b in range(B):
            ctx_cols = []
            for h in range(H_LOC):
                q = q_all[b * Sq:(b + 1) * Sq, h * Dh:(h + 1) * Dh]
                k = k_ref[b, :, h, :]
                v = v_ref[b, :, h, :]
                s = lax.dot_general(
                    q, k, (((1,), (1,)), ((), ())),
                    preferred_element_type=jnp.float32)
                w = jnp.exp(s) * mask01
                denom = jnp.sum(w, axis=1, keepdims=True)
                ctx_cols.append(
                    jnp.dot(w, v, preferred_element_type=jnp.float32)
                    / denom)
            ctx_rows.append(jnp.concatenate(ctx_cols, axis=1))
        ctx_all = jnp.concatenate(ctx_rows, axis=0)
        part_ref[...] = jnp.dot(
            ctx_all, wo_loc,
            preferred_element_type=jnp.float32).reshape(B, Sq, D_MODEL)

        def part_chunk(c, lo):
            return part_ref[:, pl.ds((c % N_DEV) * CH, CH), lo:lo + HALF]

        def put_out(c, lo, val):
            out_ref[:, pl.ds((c % N_DEV) * CH, CH), lo:lo + HALF] = (
                val.astype(jnp.float32))

        stage_r[0] = part_chunk(p, 0).astype(COMM_DT)
        stage_l[0] = part_chunk(p, HALF).astype(COMM_DT)

        def start(src, dst, ssem, rsem, s, dest):
            rdma = pltpu.make_async_remote_copy(
                src_ref=src.at[s], dst_ref=dst.at[s],
                send_sem=ssem.at[s], recv_sem=rsem.at[s],
                device_id=(dest,), device_id_type=pl.DeviceIdType.MESH,
            )
            rdma.start()
            return rdma

        for s in range(N_DEV - 1):
            rr = start(stage_r, recv_r, ssem_r, rsem_r, s, right)
            rl = start(stage_l, recv_l, ssem_l, rsem_l, s, left)
            rr.wait()
            rl.wait()
            acc_r = recv_r[s].astype(jnp.float32) + part_chunk(p - s - 1, 0)
            acc_l = recv_l[s].astype(jnp.float32) + part_chunk(p + s + 1, HALF)
            if s < N_DEV - 2:
                stage_r[s + 1] = acc_r.astype(COMM_DT)
                stage_l[s + 1] = acc_l.astype(COMM_DT)
            else:
                put_out(p + 1, 0, acc_r)
                put_out(p - 1, HALF, acc_l)
                stage_r[N_DEV - 1] = acc_r.astype(COMM_DT)
                stage_l[N_DEV - 1] = acc_l.astype(COMM_DT)

        for t in range(N_DEV - 1):
            s = (N_DEV - 1) + t
            rr = start(stage_r, recv_r, ssem_r, rsem_r, s, right)
            rl = start(stage_l, recv_l, ssem_l, rsem_l, s, left)
            rr.wait()
            rl.wait()
            put_out(p - t, 0, recv_r[s])
            put_out(p + t, HALF, recv_l[s])
            if t < N_DEV - 2:
                stage_r[s + 1] = recv_r[s]
                stage_l[s + 1] = recv_l[s]

    chunk = (B, CH, HALF)
    return pl.pallas_call(
        body,
        out_shape=jax.ShapeDtypeStruct((B, Sq, D_MODEL), jnp.float32),
        in_specs=[pl.BlockSpec(memory_space=pltpu.VMEM)] * 5,
        out_specs=pl.BlockSpec(memory_space=pltpu.VMEM),
        scratch_shapes=[
            pltpu.VMEM((B, Sq, D_MODEL), jnp.float32),
            pltpu.VMEM((N_STEP,) + chunk, COMM_DT),
            pltpu.VMEM((N_STEP,) + chunk, COMM_DT),
            pltpu.VMEM((N_STEP,) + chunk, COMM_DT),
            pltpu.VMEM((N_STEP,) + chunk, COMM_DT),
            pltpu.SemaphoreType.DMA((N_STEP,)),
            pltpu.SemaphoreType.DMA((N_STEP,)),
            pltpu.SemaphoreType.DMA((N_STEP,)),
            pltpu.SemaphoreType.DMA((N_STEP,)),
        ],
        compiler_params=pltpu.CompilerParams(collective_id=0),
    )(x, Wq, K_ext, V_ext, Wo)


# device time: 47695 ns/iter; 2.8193x vs baseline; 1.2648x over previous
import jax
import jax.numpy as jnp
from jax import lax
from jax.experimental import pallas as pl
from jax.experimental.pallas import tpu as pltpu

N_DEV = 4
B, Sq, Skv, Dh = 2, 512, 512, 64
H_LOC = 8
D_LOC = H_LOC * Dh
D_MODEL = 768
HALF = D_MODEL // 2
CH = Sq // N_DEV
WINDOW = 128
N_STEP = 2 * (N_DEV - 1)

COMM_DT = jnp.bfloat16


def kernel(x, Wq, K_ext, V_ext, Wo):
    def body(x_ref, wq_ref, k_ref, v_ref, wo_ref, out_ref, part_ref,
             stage_r, recv_r, stage_l, recv_l,
             ssem_r, rsem_r, ssem_l, rsem_l):
        p = lax.axis_index("i")
        left = (p - 1) % N_DEV
        right = (p + 1) % N_DEV

        barrier_sem = pltpu.get_barrier_semaphore()
        for nbr in [left, right]:
            pl.semaphore_signal(
                barrier_sem, inc=1,
                device_id=(nbr,), device_id_type=pl.DeviceIdType.MESH,
            )
        pl.semaphore_wait(barrier_sem, 2)

        col0 = p * D_LOC
        wq_loc = wq_ref[:, pl.ds(col0, D_LOC)]
        wo_loc = wo_ref[pl.ds(col0, D_LOC), :]

        qi = lax.broadcasted_iota(jnp.int32, (Sq, Skv), 0)
        ki = lax.broadcasted_iota(jnp.int32, (Sq, Skv), 1)
        mask01 = jnp.where(jnp.abs(qi - ki) <= WINDOW,
                           jnp.float32(1.0), jnp.float32(0.0))

        x2 = x_ref[...].reshape(B * Sq, D_MODEL)
        q_all = jnp.dot(x2, wq_loc,
                        preferred_element_type=jnp.float32) * 0.125

        ctx_rows = []
        for b in range(B):
            ctx_cols = []
            for h in range(H_LOC):
                q = q_all[b * Sq:(b + 1) * Sq, h * Dh:(h + 1) * Dh]
                k = k_ref[b, :, h, :]
                v = v_ref[b, :, h, :]
                s = lax.dot_general(
                    q, k, (((1,), (1,)), ((), ())),
                    preferred_element_type=jnp.float32)
                w = jnp.exp(s) * mask01
                denom = jnp.sum(w, axis=1, keepdims=True)
                ctx_cols.append(
                    jnp.dot(w, v, preferred_element_type=jnp.float32)
                    / denom)
            ctx_rows.append(jnp.concatenate(ctx_cols, axis=1))
        ctx_all = jnp.concatenate(ctx_rows, axis=0)
        part_ref[...] = jnp.dot(
            ctx_all, wo_loc,
            preferred_element_type=jnp.float32).reshape(B, Sq, D_MODEL)

        def part_chunk(c, lo):
            return part_ref[:, pl.ds((c % N_DEV) * CH, CH), lo:lo + HALF]

        def put_out(c, lo, val):
            out_ref[:, pl.ds((c % N_DEV) * CH, CH), lo:lo + HALF] = (
                val.astype(jnp.float32))

        stage_r[0] = part_chunk(p, 0).astype(COMM_DT)
        stage_l[0] = part_chunk(p, HALF).astype(COMM_DT)

        def start(src, dst, ssem, rsem, s, dest):
            rdma = pltpu.make_async_remote_copy(
                src_ref=src.at[s], dst_ref=dst.at[s],
                send_sem=ssem.at[s], recv_sem=rsem.at[s],
                device_id=(dest,), device_id_type=pl.DeviceIdType.MESH,
            )
            rdma.start()
            return rdma

        for s in range(N_DEV - 1):
            rr = start(stage_r, recv_r, ssem_r, rsem_r, s, right)
            rl = start(stage_l, recv_l, ssem_l, rsem_l, s, left)
            rr.wait()
            rl.wait()
            acc_r = recv_r[s].astype(jnp.float32) + part_chunk(p - s - 1, 0)
            acc_l = recv_l[s].astype(jnp.float32) + part_chunk(p + s + 1, HALF)
            if s < N_DEV - 2:
                stage_r[s + 1] = acc_r.astype(COMM_DT)
                stage_l[s + 1] = acc_l.astype(COMM_DT)
            else:
                put_out(p + 1, 0, acc_r)
                put_out(p - 1, HALF, acc_l)
                stage_r[N_DEV - 1] = acc_r.astype(COMM_DT)
                stage_l[N_DEV - 1] = acc_l.astype(COMM_DT)

        for t in range(N_DEV - 1):
            s = (N_DEV - 1) + t
            rr = start(stage_r, recv_r, ssem_r, rsem_r, s, right)
            rl = start(stage_l, recv_l, ssem_l, rsem_l, s, left)
            rr.wait()
            rl.wait()
            put_out(p - t, 0, recv_r[s])
            put_out(p + t, HALF, recv_l[s])
            if t < N_DEV - 2:
                stage_r[s + 1] = recv_r[s]
                stage_l[s + 1] = recv_l[s]

    chunk = (B, CH, HALF)
    return pl.pallas_call(
        body,
        out_shape=jax.ShapeDtypeStruct((B, Sq, D_MODEL), jnp.float32),
        in_specs=[pl.BlockSpec(memory_space=pltpu.VMEM)] * 5,
        out_specs=pl.BlockSpec(memory_space=pltpu.VMEM),
        scratch_shapes=[
            pltpu.VMEM((B, Sq, D_MODEL), jnp.float32),
            pltpu.VMEM((N_STEP,) + chunk, COMM_DT),
            pltpu.VMEM((N_STEP,) + chunk, COMM_DT),
            pltpu.VMEM((N_STEP,) + chunk, COMM_DT),
            pltpu.VMEM((N_STEP,) + chunk, COMM_DT),
            pltpu.SemaphoreType.DMA((N_STEP,)),
            pltpu.SemaphoreType.DMA((N_STEP,)),
            pltpu.SemaphoreType.DMA((N_STEP,)),
            pltpu.SemaphoreType.DMA((N_STEP,)),
        ],
        compiler_params=pltpu.CompilerParams(collective_id=0),
    )(x, Wq, K_ext, V_ext, Wo)
